# baseline (speedup 1.0000x reference)
import jax
import jax.numpy as jnp
from jax import lax
from jax.experimental import pallas as pl
from jax.experimental.pallas import tpu as pltpu
from jax.experimental.pallas import tpu_sc as plsc

D_MODEL = 1024
SEQ_LEN = 4096
BATCH = 4

NUM_WORKERS = 32
TILE_ROWS = 3072
TILE_ROWS_PER_WORKER = TILE_ROWS // NUM_WORKERS

SCS_ROWS = SEQ_LEN - TILE_ROWS
SCS_ROWS_PER_CORE = SCS_ROWS // 2
SCS_CHUNK = 256
SCS_NCHUNK = SCS_ROWS_PER_CORE // SCS_CHUNK


def _sc_broadcast(pos_table):
    vmesh = plsc.VectorSubcoreMesh(core_axis_name="c", subcore_axis_name="s")
    smesh = plsc.ScalarSubcoreMesh(axis_name="c", num_cores=2)

    def vector_body(pos_hbm, out_hbm, spmem, sc_lsem, sc_ssem, tbuf, tsem):
        del spmem, sc_lsem, sc_ssem
        wid = lax.axis_index("s") * vmesh.num_cores + lax.axis_index("c")
        base = wid * TILE_ROWS_PER_WORKER
        nsub = 3
        sub = TILE_ROWS_PER_WORKER // nsub
        loads = [
            pltpu.async_copy(
                pos_hbm.at[pl.ds(base + h * sub, sub)],
                tbuf.at[h],
                tsem.at[0],
            )
            for h in range(nsub)
        ]
        copies = []
        for h in range(nsub):
            loads[h].wait()
            copies += [
                pltpu.async_copy(
                    tbuf.at[h],
                    out_hbm.at[b, pl.ds(base + h * sub, sub)],
                    tsem.at[1],
                )
                for b in range(BATCH)
            ]
        for c in copies:
            c.wait()

    def scalar_body(pos_hbm, out_hbm, spmem, sc_lsem, sc_ssem, tbuf, tsem):
        del tbuf, tsem
        cid = lax.axis_index("c")
        base = TILE_ROWS + cid * SCS_ROWS_PER_CORE

        def load(c, slot):
            return pltpu.async_copy(
                pos_hbm.at[pl.ds(base + c * SCS_CHUNK, SCS_CHUNK)],
                spmem.at[slot],
                sc_lsem.at[slot],
            )

        def stores(c, slot):
            return [
                pltpu.async_copy(
                    spmem.at[slot],
                    out_hbm.at[b, pl.ds(base + c * SCS_CHUNK, SCS_CHUNK)],
                    sc_ssem.at[slot],
                )
                for b in range(BATCH)
            ]

        loads = [load(0, 0), load(1, 1)]
        pending = [None, None]
        for c in range(SCS_NCHUNK):
            slot = c % 2
            loads[slot].wait()
            sts = stores(c, slot)
            nxt = c + 2
            if nxt < SCS_NCHUNK:
                for cp in sts:
                    cp.wait()
                loads[slot] = load(nxt, slot)
            else:
                pending[slot] = sts
        for group in pending:
            if group is not None:
                for cp in group:
                    cp.wait()

    run = pl.kernel(
        [scalar_body, vector_body],
        mesh=[smesh, vmesh],
        out_type=jax.ShapeDtypeStruct((BATCH, SEQ_LEN, D_MODEL), jnp.float32),
        scratch_types=[
            pltpu.VMEM_SHARED((2, SCS_CHUNK, D_MODEL), jnp.float32),
            pltpu.SemaphoreType.DMA((2,)) @ smesh,
            pltpu.SemaphoreType.DMA((2,)) @ smesh,
            pltpu.VMEM((3, TILE_ROWS_PER_WORKER // 3, D_MODEL), jnp.float32)
            @ vmesh,
            pltpu.SemaphoreType.DMA((2,)) @ vmesh,
        ],
    )
    return run(pos_table)


def kernel(x, pos_table):
    del x
    return _sc_broadcast(pos_table)


# speedup vs baseline: 1.0046x; 1.0046x over previous
import jax
import jax.numpy as jnp
from jax import lax
from jax.experimental import pallas as pl
from jax.experimental.pallas import tpu as pltpu
from jax.experimental.pallas import tpu_sc as plsc

D_MODEL = 1024
SEQ_LEN = 4096
BATCH = 4

NUM_WORKERS = 32
TILE_ROWS = 3072
TILE_ROWS_PER_WORKER = TILE_ROWS // NUM_WORKERS

SCS_ROWS = SEQ_LEN - TILE_ROWS
SCS_ROWS_PER_CORE = SCS_ROWS // 2
SCS_CHUNK = 256
SCS_NCHUNK = SCS_ROWS_PER_CORE // SCS_CHUNK


def _sc_broadcast(pos_table):
    vmesh = plsc.VectorSubcoreMesh(core_axis_name="c", subcore_axis_name="s")
    smesh = plsc.ScalarSubcoreMesh(axis_name="c", num_cores=2)

    def vector_body(pos_hbm, out_hbm, spmem, sc_lsem, sc_ssem, tbuf, tsem):
        del spmem, sc_lsem, sc_ssem
        wid = lax.axis_index("s") * vmesh.num_cores + lax.axis_index("c")
        base = wid * TILE_ROWS_PER_WORKER
        nsub = 2
        sub = TILE_ROWS_PER_WORKER // nsub
        loads = [
            pltpu.async_copy(
                pos_hbm.at[pl.ds(base + h * sub, sub)],
                tbuf.at[h],
                tsem.at[0],
            )
            for h in range(nsub)
        ]
        copies = []
        for h in range(nsub):
            loads[h].wait()
            copies += [
                pltpu.async_copy(
                    tbuf.at[h],
                    out_hbm.at[b, pl.ds(base + h * sub, sub)],
                    tsem.at[1],
                )
                for b in range(BATCH)
            ]
        for c in copies:
            c.wait()

    def scalar_body(pos_hbm, out_hbm, spmem, sc_lsem, sc_ssem, tbuf, tsem):
        del tbuf, tsem
        cid = lax.axis_index("c")
        base = TILE_ROWS + cid * SCS_ROWS_PER_CORE

        def load(c, slot):
            return pltpu.async_copy(
                pos_hbm.at[pl.ds(base + c * SCS_CHUNK, SCS_CHUNK)],
                spmem.at[slot],
                sc_lsem.at[slot],
            )

        def stores(c, slot):
            return [
                pltpu.async_copy(
                    spmem.at[slot],
                    out_hbm.at[b, pl.ds(base + c * SCS_CHUNK, SCS_CHUNK)],
                    sc_ssem.at[slot],
                )
                for b in range(BATCH)
            ]

        loads = [load(0, 0), load(1, 1)]
        pending = [None, None]
        for c in range(SCS_NCHUNK):
            slot = c % 2
            loads[slot].wait()
            sts = stores(c, slot)
            nxt = c + 2
            if nxt < SCS_NCHUNK:
                for cp in sts:
                    cp.wait()
                loads[slot] = load(nxt, slot)
            else:
                pending[slot] = sts
        for group in pending:
            if group is not None:
                for cp in group:
                    cp.wait()

    run = pl.kernel(
        [scalar_body, vector_body],
        mesh=[smesh, vmesh],
        out_type=jax.ShapeDtypeStruct((BATCH, SEQ_LEN, D_MODEL), jnp.float32),
        scratch_types=[
            pltpu.VMEM_SHARED((2, SCS_CHUNK, D_MODEL), jnp.float32),
            pltpu.SemaphoreType.DMA((2,)) @ smesh,
            pltpu.SemaphoreType.DMA((2,)) @ smesh,
            pltpu.VMEM((2, TILE_ROWS_PER_WORKER // 2, D_MODEL), jnp.float32)
            @ vmesh,
            pltpu.SemaphoreType.DMA((2,)) @ vmesh,
        ],
    )
    return run(pos_table)


def kernel(x, pos_table):
    del x
    return _sc_broadcast(pos_table)
